# baseline (speedup 1.0000x reference)
import functools

import jax
import jax.numpy as jnp
from jax import lax
from jax.experimental import pallas as pl
from jax.experimental.pallas import tpu as pltpu
from jax.experimental.pallas import tpu_sc as plsc

B = 16384
V = 100000
D_STATION = 16
D_MEMBER = 4
N_NUM = 16
HIDDEN = 128
IN_DIM = 2 * D_STATION + D_MEMBER + N_NUM
XT_ROWS = 56

_INFO = plsc.get_sparse_core_info()
_NC = _INFO.num_cores
_NS = _INFO.num_subcores
_NW = _NC * _NS
_BPW = B // _NW
_CHUNK = 128
_NCHUNK = _BPW // _CHUNK
_ROWS = B // _CHUNK
_L = 16


_OFF_E = V * D_STATION // 128
_OFF_M = 2 * _OFF_E


def _sc_gather_body(idx0_hbm, idx1_hbm, idx2_hbm, ts_hbm, te_hbm, tm_hbm,
                    xn_hbm, out_xt,
                    iv0, iv1, iv2, dv2, bs, be, bm, ov,
                    sem_s, sem_e, sem_m, sem_x):
    wid = lax.axis_index("s") * _NC + lax.axis_index("c")
    base = wid * _BPW
    r0 = wid * _NCHUNK
    pltpu.sync_copy(idx0_hbm.at[pl.ds(r0, _NCHUNK)], iv0)
    pltpu.sync_copy(idx1_hbm.at[pl.ds(r0, _NCHUNK)], iv1)
    pltpu.sync_copy(idx2_hbm.at[pl.ds(r0, _NCHUNK)], iv2)
    xn_cp = pltpu.async_copy(xn_hbm.at[:, pl.ds(base, _BPW)],
                             ov.at[pl.ds(36, N_NUM)], sem_x)

    for c in range(_NCHUNK):
        for g in range(_CHUNK // _L):
            sl = (c, pl.ds(g * _L, _L))
            dv2[sl] = jax.lax.shift_right_logical(iv2[sl], 2)

    zv = jnp.zeros((_L,), jnp.float32)
    for j in range(IN_DIM, XT_ROWS):
        for g in range(_BPW // _L):
            ov[j, pl.ds(g * _L, _L)] = zv

    iot = lax.iota(jnp.int32, _L)

    def fire(c, bufs):
        bs_c, be_c = bufs
        return [
            pltpu.async_copy(ts_hbm.at[iv0.at[c]], bs_c, sem_s),
            pltpu.async_copy(te_hbm.at[iv1.at[c]], be_c, sem_e),
        ]

    def select(c, bufs, bm_c):
        bs_c, be_c = bufs
        for g in range(_CHUNK // _L):
            sl = (c, pl.ds(g * _L, _L))
            rows = g * _L + iot
            ocol = c * _CHUNK + g * _L + iot
            c2 = jnp.left_shift(jnp.bitwise_and(iv2[sl], 3), 2)
            for j in range(D_STATION):
                js = jnp.full((_L,), j, jnp.int32)
                v0 = plsc.load_gather(bs_c, [rows, js])
                plsc.store_scatter(ov, [js, ocol], v0)
                v1 = plsc.load_gather(be_c, [rows, js])
                plsc.store_scatter(ov, [js + D_STATION, ocol], v1)
            for j in range(D_MEMBER):
                js = jnp.full((_L,), 2 * D_STATION + j, jnp.int32)
                v2 = plsc.load_gather(bm_c, [rows, c2 + j])
                plsc.store_scatter(ov, [js, ocol], v2)

    bufs = [(bs.at[0], be.at[0]), (bs.at[1], be.at[1])]
    pend = fire(0, bufs[0])
    for c in range(_NCHUNK):
        bm_cp = pltpu.async_copy(tm_hbm.at[dv2.at[c]], bm, sem_m)
        for cp in pend:
            cp.wait()
        if c + 1 < _NCHUNK:
            nxt = fire(c + 1, bufs[(c + 1) % 2])
        else:
            nxt = []
        bm_cp.wait()
        select(c, bufs[c % 2], bm)
        pend = nxt

    xn_cp.wait()
    pltpu.sync_copy(ov, out_xt.at[:, pl.ds(base, _BPW)])


_sc_gather = functools.partial(
    pl.kernel,
    mesh=plsc.VectorSubcoreMesh(core_axis_name="c", subcore_axis_name="s"),
    out_type=jax.ShapeDtypeStruct((XT_ROWS, B), jnp.float32),
    scratch_types=[
        pltpu.VMEM((_NCHUNK, _CHUNK), jnp.int32),
        pltpu.VMEM((_NCHUNK, _CHUNK), jnp.int32),
        pltpu.VMEM((_NCHUNK, _CHUNK), jnp.int32),
        pltpu.VMEM((_NCHUNK, _CHUNK), jnp.int32),
        pltpu.VMEM((2, _CHUNK, D_STATION), jnp.float32),
        pltpu.VMEM((2, _CHUNK, D_STATION), jnp.float32),
        pltpu.VMEM((_CHUNK, 16), jnp.float32),
        pltpu.VMEM((XT_ROWS, _BPW), jnp.float32),
        pltpu.SemaphoreType.DMA,
        pltpu.SemaphoreType.DMA,
        pltpu.SemaphoreType.DMA,
        pltpu.SemaphoreType.DMA,
    ],
    compiler_params=pltpu.CompilerParams(use_tc_tiling_on_sc=False,
                                         needs_layout_passes=False),
)(_sc_gather_body)


def _mlp_body(xt_ref, w1, b1, w2, b2, w3, b3, out_ref):
    f32 = jnp.float32
    dn = (((0,), (0,)), ((), ()))
    x = xt_ref[...]
    h = lax.dot_general(w1[...], x, dn, preferred_element_type=f32) + b1[...]
    h = jnp.maximum(h, 0.0)
    h = lax.dot_general(w2[...], h, dn, preferred_element_type=f32) + b2[...]
    h = jnp.maximum(h, 0.0)
    out_ref[...] = (lax.dot_general(w3[...], h, dn, preferred_element_type=f32)
                    + b3[...])


_BLK = 2048


def _mlp(xt, w1, b1, w2, b2, w3, b3):
    nblk = B // _BLK
    full = lambda shape: pl.BlockSpec(shape, lambda i: (0, 0))
    return pl.pallas_call(
        _mlp_body,
        grid=(nblk,),
        in_specs=[
            pl.BlockSpec((XT_ROWS, _BLK), lambda i: (0, i)),
            full((XT_ROWS, HIDDEN)), full((HIDDEN, 1)),
            full((HIDDEN, HIDDEN)), full((HIDDEN, 1)),
            full((HIDDEN, 2)), full((2, 1)),
        ],
        out_specs=pl.BlockSpec((2, _BLK), lambda i: (0, i)),
        out_shape=jax.ShapeDtypeStruct((2, B), jnp.float32),
    )(xt, w1, b1, w2, b2, w3, b3)


def kernel(x_cat, x_num, emb_start, emb_end, emb_member, W1, b1, W2, b2, W3, b3):
    idx0 = x_cat[:, 0].reshape(_ROWS, _CHUNK)
    idx1 = x_cat[:, 1].reshape(_ROWS, _CHUNK)
    idx2 = x_cat[:, 2].reshape(_ROWS, _CHUNK)
    tm2 = emb_member.reshape(V * D_MEMBER // 16, 16)
    xt = _sc_gather(idx0, idx1, idx2, emb_start, emb_end, tm2, x_num.T)
    w1z = jnp.pad(W1, ((0, XT_ROWS - IN_DIM), (0, 0)))
    out = _mlp(xt, w1z, b1.reshape(HIDDEN, 1), W2, b2.reshape(HIDDEN, 1),
               W3, b3.reshape(2, 1))
    return out[0], out[1]


# speedup vs baseline: 1.0117x; 1.0117x over previous
import functools

import jax
import jax.numpy as jnp
from jax import lax
from jax.experimental import pallas as pl
from jax.experimental.pallas import tpu as pltpu
from jax.experimental.pallas import tpu_sc as plsc

B = 16384
V = 100000
D_STATION = 16
D_MEMBER = 4
N_NUM = 16
HIDDEN = 128
IN_DIM = 2 * D_STATION + D_MEMBER + N_NUM
XT_ROWS = 56

_INFO = plsc.get_sparse_core_info()
_NC = _INFO.num_cores
_NS = _INFO.num_subcores
_NW = _NC * _NS
_BPW = B // _NW
_CHUNK = 128
_NCHUNK = _BPW // _CHUNK
_ROWS = B // _CHUNK
_L = 16


_OFF_E = V * D_STATION // 128
_OFF_M = 2 * _OFF_E


def _sc_gather_body(idx0_hbm, idx1_hbm, idx2_hbm, ts_hbm, te_hbm, tm_hbm,
                    xn_hbm, out_xt,
                    iv0, iv1, iv2, dv2, bs, be, bm, ov,
                    sem_s, sem_e, sem_m, sem_x):
    wid = lax.axis_index("s") * _NC + lax.axis_index("c")
    base = wid * _BPW
    r0 = wid * _NCHUNK
    pltpu.sync_copy(idx0_hbm.at[pl.ds(r0, _NCHUNK)], iv0)
    pltpu.sync_copy(idx1_hbm.at[pl.ds(r0, _NCHUNK)], iv1)
    pltpu.sync_copy(idx2_hbm.at[pl.ds(r0, _NCHUNK)], iv2)
    xn_cp = pltpu.async_copy(xn_hbm.at[:, pl.ds(base, _BPW)],
                             ov.at[pl.ds(36, N_NUM)], sem_x)

    for c in range(_NCHUNK):
        for g in range(_CHUNK // _L):
            sl = (c, pl.ds(g * _L, _L))
            dv2[sl] = jax.lax.shift_right_logical(iv2[sl], 2)

    zv = jnp.zeros((_L,), jnp.float32)
    for j in range(IN_DIM, XT_ROWS):
        for g in range(_BPW // _L):
            ov[j, pl.ds(g * _L, _L)] = zv

    iot = lax.iota(jnp.int32, _L)

    def fire(c, bufs):
        bs_c, be_c = bufs
        return [
            pltpu.async_copy(ts_hbm.at[iv0.at[c]], bs_c, sem_s),
            pltpu.async_copy(te_hbm.at[iv1.at[c]], be_c, sem_e),
        ]

    def select(c, bufs, bm_c):
        bs_c, be_c = bufs
        for g in range(_CHUNK // _L):
            sl = (c, pl.ds(g * _L, _L))
            rows = g * _L + iot
            ocol = c * _CHUNK + g * _L + iot
            c2 = jnp.left_shift(jnp.bitwise_and(iv2[sl], 3), 2)
            for j in range(D_STATION):
                js = jnp.full((_L,), j, jnp.int32)
                v0 = plsc.load_gather(bs_c, [rows, js])
                plsc.store_scatter(ov, [js, ocol], v0)
                v1 = plsc.load_gather(be_c, [rows, js])
                plsc.store_scatter(ov, [js + D_STATION, ocol], v1)
            for j in range(D_MEMBER):
                js = jnp.full((_L,), 2 * D_STATION + j, jnp.int32)
                v2 = plsc.load_gather(bm_c, [rows, c2 + j])
                plsc.store_scatter(ov, [js, ocol], v2)

    bufs = [(bs.at[0], be.at[0]), (bs.at[1], be.at[1])]
    pend = fire(0, bufs[0])
    for c in range(_NCHUNK):
        bm_cp = pltpu.async_copy(tm_hbm.at[dv2.at[c]], bm, sem_m)
        for cp in pend:
            cp.wait()
        if c + 1 < _NCHUNK:
            nxt = fire(c + 1, bufs[(c + 1) % 2])
        else:
            nxt = []
        bm_cp.wait()
        select(c, bufs[c % 2], bm)
        pend = nxt

    xn_cp.wait()
    pltpu.sync_copy(ov, out_xt.at[:, pl.ds(base, _BPW)])


_sc_gather = functools.partial(
    pl.kernel,
    mesh=plsc.VectorSubcoreMesh(core_axis_name="c", subcore_axis_name="s"),
    out_type=jax.ShapeDtypeStruct((XT_ROWS, B), jnp.float32),
    scratch_types=[
        pltpu.VMEM((_NCHUNK, _CHUNK), jnp.int32),
        pltpu.VMEM((_NCHUNK, _CHUNK), jnp.int32),
        pltpu.VMEM((_NCHUNK, _CHUNK), jnp.int32),
        pltpu.VMEM((_NCHUNK, _CHUNK), jnp.int32),
        pltpu.VMEM((2, _CHUNK, D_STATION), jnp.float32),
        pltpu.VMEM((2, _CHUNK, D_STATION), jnp.float32),
        pltpu.VMEM((_CHUNK, 16), jnp.float32),
        pltpu.VMEM((XT_ROWS, _BPW), jnp.float32),
        pltpu.SemaphoreType.DMA,
        pltpu.SemaphoreType.DMA,
        pltpu.SemaphoreType.DMA,
        pltpu.SemaphoreType.DMA,
    ],
    compiler_params=pltpu.CompilerParams(use_tc_tiling_on_sc=False,
                                         needs_layout_passes=False),
)(_sc_gather_body)


def _mlp_body(xt_ref, w1, b1, w2, b2, w3, b3, out_ref):
    f32 = jnp.float32
    dn = (((0,), (0,)), ((), ()))
    x = xt_ref[...]
    h = lax.dot_general(w1[...], x, dn, preferred_element_type=f32) + b1[...]
    h = jnp.maximum(h, 0.0)
    h = lax.dot_general(w2[...], h, dn, preferred_element_type=f32) + b2[...]
    h = jnp.maximum(h, 0.0)
    out_ref[...] = (lax.dot_general(w3[...], h, dn, preferred_element_type=f32)
                    + b3[...])


_BLK = 4096


def _mlp(xt, w1, b1, w2, b2, w3, b3):
    nblk = B // _BLK
    full = lambda shape: pl.BlockSpec(shape, lambda i: (0, 0))
    return pl.pallas_call(
        _mlp_body,
        grid=(nblk,),
        in_specs=[
            pl.BlockSpec((XT_ROWS, _BLK), lambda i: (0, i)),
            full((XT_ROWS, HIDDEN)), full((HIDDEN, 1)),
            full((HIDDEN, HIDDEN)), full((HIDDEN, 1)),
            full((HIDDEN, 2)), full((2, 1)),
        ],
        out_specs=pl.BlockSpec((2, _BLK), lambda i: (0, i)),
        out_shape=jax.ShapeDtypeStruct((2, B), jnp.float32),
    )(xt, w1, b1, w2, b2, w3, b3)


def kernel(x_cat, x_num, emb_start, emb_end, emb_member, W1, b1, W2, b2, W3, b3):
    idx0 = x_cat[:, 0].reshape(_ROWS, _CHUNK)
    idx1 = x_cat[:, 1].reshape(_ROWS, _CHUNK)
    idx2 = x_cat[:, 2].reshape(_ROWS, _CHUNK)
    tm2 = emb_member.reshape(V * D_MEMBER // 16, 16)
    xt = _sc_gather(idx0, idx1, idx2, emb_start, emb_end, tm2, x_num.T)
    w1z = jnp.pad(W1, ((0, XT_ROWS - IN_DIM), (0, 0)))
    out = _mlp(xt, w1z, b1.reshape(HIDDEN, 1), W2, b2.reshape(HIDDEN, 1),
               W3, b3.reshape(2, 1))
    return out[0], out[1]


# speedup vs baseline: 1.0136x; 1.0019x over previous
import functools

import jax
import jax.numpy as jnp
from jax import lax
from jax.experimental import pallas as pl
from jax.experimental.pallas import tpu as pltpu
from jax.experimental.pallas import tpu_sc as plsc

B = 16384
V = 100000
D_STATION = 16
D_MEMBER = 4
N_NUM = 16
HIDDEN = 128
IN_DIM = 2 * D_STATION + D_MEMBER + N_NUM
XT_ROWS = 56

_INFO = plsc.get_sparse_core_info()
_NC = _INFO.num_cores
_NS = _INFO.num_subcores
_NW = _NC * _NS
_BPW = B // _NW
_CHUNK = 128
_NCHUNK = _BPW // _CHUNK
_ROWS = B // _CHUNK
_L = 16


_OFF_E = V * D_STATION // 128
_OFF_M = 2 * _OFF_E


def _sc_gather_body(idx0_hbm, idx1_hbm, idx2_hbm, ts_hbm, te_hbm, tm_hbm,
                    xn_hbm, out_xt,
                    iv0, iv1, iv2, dv2, bs, be, bm, ov,
                    sem_s, sem_e, sem_m, sem_x):
    wid = lax.axis_index("s") * _NC + lax.axis_index("c")
    base = wid * _BPW
    r0 = wid * _NCHUNK
    pltpu.sync_copy(idx0_hbm.at[pl.ds(r0, _NCHUNK)], iv0)
    pltpu.sync_copy(idx1_hbm.at[pl.ds(r0, _NCHUNK)], iv1)
    pltpu.sync_copy(idx2_hbm.at[pl.ds(r0, _NCHUNK)], iv2)
    xn_cp = pltpu.async_copy(xn_hbm.at[:, pl.ds(base, _BPW)],
                             ov.at[pl.ds(36, N_NUM)], sem_x)

    for c in range(_NCHUNK):
        for g in range(_CHUNK // _L):
            sl = (c, pl.ds(g * _L, _L))
            dv2[sl] = jax.lax.shift_right_logical(iv2[sl], 2)

    zv = jnp.zeros((_L,), jnp.float32)
    for j in range(IN_DIM, XT_ROWS):
        for g in range(_BPW // _L):
            ov[j, pl.ds(g * _L, _L)] = zv

    iot = lax.iota(jnp.int32, _L)

    def fire(c, bufs):
        bs_c, be_c = bufs
        return [
            pltpu.async_copy(ts_hbm.at[iv0.at[c]], bs_c, sem_s),
            pltpu.async_copy(te_hbm.at[iv1.at[c]], be_c, sem_e),
        ]

    def select(c, bufs, bm_c):
        bs_c, be_c = bufs
        for g in range(_CHUNK // _L):
            sl = (c, pl.ds(g * _L, _L))
            rows = g * _L + iot
            ocol = c * _CHUNK + g * _L + iot
            c2 = jnp.left_shift(jnp.bitwise_and(iv2[sl], 3), 2)
            for j in range(D_STATION):
                js = jnp.full((_L,), j, jnp.int32)
                v0 = plsc.load_gather(bs_c, [rows, js])
                plsc.store_scatter(ov, [js, ocol], v0)
                v1 = plsc.load_gather(be_c, [rows, js])
                plsc.store_scatter(ov, [js + D_STATION, ocol], v1)
            for j in range(D_MEMBER):
                js = jnp.full((_L,), 2 * D_STATION + j, jnp.int32)
                v2 = plsc.load_gather(bm_c, [rows, c2 + j])
                plsc.store_scatter(ov, [js, ocol], v2)

    bufs = [(bs.at[0], be.at[0]), (bs.at[1], be.at[1])]
    pend = fire(0, bufs[0])
    for c in range(_NCHUNK):
        bm_cp = pltpu.async_copy(tm_hbm.at[dv2.at[c]], bm, sem_m)
        for cp in pend:
            cp.wait()
        if c + 1 < _NCHUNK:
            nxt = fire(c + 1, bufs[(c + 1) % 2])
        else:
            nxt = []
        bm_cp.wait()
        select(c, bufs[c % 2], bm)
        pend = nxt

    xn_cp.wait()
    pltpu.sync_copy(ov, out_xt.at[:, pl.ds(base, _BPW)])


_sc_gather = functools.partial(
    pl.kernel,
    mesh=plsc.VectorSubcoreMesh(core_axis_name="c", subcore_axis_name="s"),
    out_type=jax.ShapeDtypeStruct((XT_ROWS, B), jnp.float32),
    scratch_types=[
        pltpu.VMEM((_NCHUNK, _CHUNK), jnp.int32),
        pltpu.VMEM((_NCHUNK, _CHUNK), jnp.int32),
        pltpu.VMEM((_NCHUNK, _CHUNK), jnp.int32),
        pltpu.VMEM((_NCHUNK, _CHUNK), jnp.int32),
        pltpu.VMEM((2, _CHUNK, D_STATION), jnp.float32),
        pltpu.VMEM((2, _CHUNK, D_STATION), jnp.float32),
        pltpu.VMEM((_CHUNK, 16), jnp.float32),
        pltpu.VMEM((XT_ROWS, _BPW), jnp.float32),
        pltpu.SemaphoreType.DMA,
        pltpu.SemaphoreType.DMA,
        pltpu.SemaphoreType.DMA,
        pltpu.SemaphoreType.DMA,
    ],
    compiler_params=pltpu.CompilerParams(use_tc_tiling_on_sc=False,
                                         needs_layout_passes=False),
)(_sc_gather_body)


def _mlp_body(xt_ref, w1, b1, w2, b2, w3, b3, out_ref):
    f32 = jnp.float32
    dn = (((0,), (0,)), ((), ()))
    x = xt_ref[...]
    h = lax.dot_general(w1[...], x, dn, preferred_element_type=f32) + b1[...]
    h = jnp.maximum(h, 0.0)
    h = lax.dot_general(w2[...], h, dn, preferred_element_type=f32) + b2[...]
    h = jnp.maximum(h, 0.0)
    out_ref[...] = (lax.dot_general(w3[...], h, dn, preferred_element_type=f32)
                    + b3[...])


_BLK = 8192


def _mlp(xt, w1, b1, w2, b2, w3, b3):
    nblk = B // _BLK
    full = lambda shape: pl.BlockSpec(shape, lambda i: (0, 0))
    return pl.pallas_call(
        _mlp_body,
        grid=(nblk,),
        in_specs=[
            pl.BlockSpec((XT_ROWS, _BLK), lambda i: (0, i)),
            full((XT_ROWS, HIDDEN)), full((HIDDEN, 1)),
            full((HIDDEN, HIDDEN)), full((HIDDEN, 1)),
            full((HIDDEN, 2)), full((2, 1)),
        ],
        out_specs=pl.BlockSpec((2, _BLK), lambda i: (0, i)),
        out_shape=jax.ShapeDtypeStruct((2, B), jnp.float32),
    )(xt, w1, b1, w2, b2, w3, b3)


def kernel(x_cat, x_num, emb_start, emb_end, emb_member, W1, b1, W2, b2, W3, b3):
    idx0 = x_cat[:, 0].reshape(_ROWS, _CHUNK)
    idx1 = x_cat[:, 1].reshape(_ROWS, _CHUNK)
    idx2 = x_cat[:, 2].reshape(_ROWS, _CHUNK)
    tm2 = emb_member.reshape(V * D_MEMBER // 16, 16)
    xt = _sc_gather(idx0, idx1, idx2, emb_start, emb_end, tm2, x_num.T)
    w1z = jnp.pad(W1, ((0, XT_ROWS - IN_DIM), (0, 0)))
    out = _mlp(xt, w1z, b1.reshape(HIDDEN, 1), W2, b2.reshape(HIDDEN, 1),
               W3, b3.reshape(2, 1))
    return out[0], out[1]
